# baseline (speedup 1.0000x reference)
import jax
import jax.numpy as jnp
from jax.experimental import pallas as pl
from jax.experimental.pallas import tpu as pltpu

D = 4
R = 6
C = 16
P = 2 * R + 1
N = P * P
DIL = 15
HH = 56
WW = 56
CF = 64
NREF = 2
W2 = 2 * WW
CH = CF // 2
WIN = 64
ROWS = HH + 2 * R + 4
OFF = 8

DXG = ((0, 4), (4, 8), (8, 12), (12, 13))


def _colorizer_body(maskadd_ref, ftpk_ref, frp_ref, labp_ref, out_ref,
                    ftpad_ref, fts_ref, frxp_ref, labm_ref, corr_ref,
                    eu_ref, acc_ref, z_ref, tmp_ref):
    ftpad_ref[...] = jnp.zeros((CH, ROWS, W2), dtype=jnp.float32)
    ftpad_ref[:, OFF:OFF + HH, :] = ftpk_ref[...]

    for dx in range(P):
        sh = labp_ref[:, :, dx:dx + WW]
        for c in range(C):
            m0 = (sh[0] == c).astype(jnp.float32)
            m1 = (sh[1] == c).astype(jnp.float32)
            labm_ref[dx, c] = jnp.concatenate((m0, m1), axis=1)

    for i in range(NREF):
        for dx in range(P):
            src = frp_ref[i, :, :, dx:dx + WW]
            frxp_ref[dx] = jnp.concatenate(
                (src[:CH], src[CH:CF]), axis=2)

        madd = maskadd_ref[i, 0, 0]

        def corr_body(dy, _, i=i, madd=madd):
            base = pl.multiple_of(jnp.where(dy >= OFF, OFF, 0), OFF)
            start = OFF + base - dy
            fts_ref[...] = ftpad_ref[:, pl.ds(start, WIN), :]
            for g0, g1 in DXG:
                accs = [None] * (g1 - g0)
                for cc in range(CH):
                    f = fts_ref[cc]
                    for j, dx in enumerate(range(g0, g1)):
                        t = f * frxp_ref[dx, cc, pl.ds(base, WIN), :]
                        accs[j] = t if cc == 0 else accs[j] + t
                for j, dx in enumerate(range(g0, g1)):
                    res = accs[j]
                    fold = res[:, :WW] + res[:, WW:] + madd
                    corr_ref[dy * P + dx, :, i * WW:(i + 1) * WW] = fold
            return 0

        jax.lax.fori_loop(0, P, corr_body, 0, unroll=False)

    acc_ref[...] = jnp.zeros((C, HH, W2), dtype=jnp.float32)
    z_ref[0] = jnp.zeros((HH, W2), dtype=jnp.float32)

    def acc_body(dy, _):
        base = pl.multiple_of(jnp.where(dy >= OFF, OFF, 0), OFF)
        o = dy - base
        zsum = None
        for dx in range(P):
            e = jnp.exp(corr_ref[dy * P + dx])
            eu_ref[dx] = e
            zsum = e if dx == 0 else zsum + e
        tmp_ref[...] = zsum
        z_ref[0] += tmp_ref[pl.ds(o, HH), :]
        for c in range(C):
            s = None
            for dx in range(P):
                t = eu_ref[dx] * labm_ref[dx, c, pl.ds(base, WIN), :]
                s = t if dx == 0 else s + t
            tmp_ref[...] = s
            acc_ref[c] += tmp_ref[pl.ds(o, HH), :]
        return 0

    jax.lax.fori_loop(0, P, acc_body, 0, unroll=False)

    z = z_ref[0]
    inv = 1.0 / (z[:, :WW] + z[:, WW:])
    for c in range(C):
        a = acc_ref[c]
        out_ref[0, c] = (a[:, :WW] + a[:, WW:]) * inv


def kernel(feats_r, feats_t, quantized_r, ref_index, current_ind):
    nref, b, cf, hh, ww = feats_r.shape
    ft = feats_t[0]
    ftpk = jnp.concatenate((ft[:CH], ft[CH:]), axis=2)
    fr = feats_r[:, 0]
    frp = jnp.pad(fr, ((0, 0), (0, 0), (R, R + 4), (R, R)))
    q = quantized_r[:, 0, 0, ::D, ::D].astype(jnp.int32)
    labp = jnp.pad(q, ((0, 0), (R, R + 4), (R, R)),
                   constant_values=-1)

    nsearch = jnp.sum((current_ind - ref_index) > DIL)
    keep = jnp.arange(nref) >= nsearch
    maskadd = jnp.where(keep, 0.0, -jnp.inf).astype(jnp.float32)
    maskadd = maskadd.reshape(nref, 1, 1)

    out = pl.pallas_call(
        _colorizer_body,
        out_shape=jax.ShapeDtypeStruct((1, C, hh, ww), jnp.float32),
        scratch_shapes=[
            pltpu.VMEM((CH, ROWS, W2), jnp.float32),
            pltpu.VMEM((CH, WIN, W2), jnp.float32),
            pltpu.VMEM((P, CH, ROWS, W2), jnp.float32),
            pltpu.VMEM((P, C, ROWS, W2), jnp.float32),
            pltpu.VMEM((N, WIN, W2), jnp.float32),
            pltpu.VMEM((P, WIN, W2), jnp.float32),
            pltpu.VMEM((C, HH, W2), jnp.float32),
            pltpu.VMEM((1, HH, W2), jnp.float32),
            pltpu.VMEM((WIN, W2), jnp.float32),
        ],
    )(maskadd, ftpk, frp, labp)
    return out


# speedup vs baseline: 1.1065x; 1.1065x over previous
import jax
import jax.numpy as jnp
from jax.experimental import pallas as pl
from jax.experimental.pallas import tpu as pltpu

D = 4
R = 6
C = 16
P = 2 * R + 1
N = P * P
DIL = 15
HH = 56
WW = 56
CF = 64
NREF = 2
W2 = 2 * WW
CH = CF // 2
WIN = 64
ROWS = HH + 2 * R + 4
OFF = 8

DXG = ((0, 4), (4, 8), (8, 12), (12, 13))


def _colorizer_body(maskadd_ref, ftpk_ref, frp_ref, labp_ref, out_ref,
                    ftpad_ref, ftsall_ref, frxp_ref, labm_ref, corr_ref,
                    eu_ref, acc_ref, z_ref, tmp_ref):
    ftpad_ref[...] = jnp.zeros((CH, ROWS, W2), dtype=jnp.float32)
    ftpad_ref[:, OFF:OFF + HH, :] = ftpk_ref[...]
    for s in range(1, OFF + 1):
        ftsall_ref[s - 1] = ftpad_ref[:, s:s + WIN, :]

    for dx in range(P):
        sh = labp_ref[:, :, dx:dx + WW]
        for c in range(C):
            labm_ref[dx, c, :, :WW] = (sh[0] == c).astype(jnp.float32)
            labm_ref[dx, c, :, WW:] = (sh[1] == c).astype(jnp.float32)

    for i in range(NREF):
        for dx in range(P):
            frxp_ref[dx, :, :, :WW] = frp_ref[i, :CH, :, dx:dx + WW]
            frxp_ref[dx, :, :, WW:] = frp_ref[i, CH:CF, :, dx:dx + WW]

        madd = maskadd_ref[i, 0, 0]

        def corr_body(dy, _, i=i, madd=madd, base=0):
            sidx = OFF - 1 + base - dy
            for g0, g1 in DXG:
                accs = [None] * (g1 - g0)
                for cc in range(CH):
                    f = ftsall_ref[sidx, cc]
                    for j, dx in enumerate(range(g0, g1)):
                        t = f * frxp_ref[dx, cc, base:base + WIN, :]
                        accs[j] = t if cc == 0 else accs[j] + t
                for j, dx in enumerate(range(g0, g1)):
                    res = accs[j]
                    fold = res[:, :WW] + res[:, WW:] + madd
                    corr_ref[dy * P + dx, :, i * WW:(i + 1) * WW] = fold
            return 0

        from functools import partial
        jax.lax.fori_loop(0, OFF, partial(corr_body, i=i, madd=madd, base=0),
                          0, unroll=False)
        jax.lax.fori_loop(OFF, P, partial(corr_body, i=i, madd=madd, base=OFF),
                          0, unroll=False)

    acc_ref[...] = jnp.zeros((C, HH, W2), dtype=jnp.float32)
    z_ref[0] = jnp.zeros((HH, W2), dtype=jnp.float32)

    def acc_body(dy, _, base=0):
        o = dy - base
        zsum = None
        for dx in range(P):
            e = jnp.exp(corr_ref[dy * P + dx])
            eu_ref[dx] = e
            zsum = e if dx == 0 else zsum + e
        tmp_ref[...] = zsum
        z_ref[0] += tmp_ref[pl.ds(o, HH), :]
        for c in range(C):
            s = None
            for dx in range(P):
                t = eu_ref[dx] * labm_ref[dx, c, base:base + WIN, :]
                s = t if dx == 0 else s + t
            tmp_ref[...] = s
            acc_ref[c] += tmp_ref[pl.ds(o, HH), :]
        return 0

    from functools import partial as _pt
    jax.lax.fori_loop(0, OFF, _pt(acc_body, base=0), 0, unroll=False)
    jax.lax.fori_loop(OFF, P, _pt(acc_body, base=OFF), 0, unroll=False)

    z = z_ref[0]
    inv = 1.0 / (z[:, :WW] + z[:, WW:])
    for c in range(C):
        a = acc_ref[c]
        out_ref[0, c] = (a[:, :WW] + a[:, WW:]) * inv


def kernel(feats_r, feats_t, quantized_r, ref_index, current_ind):
    nref, b, cf, hh, ww = feats_r.shape
    ft = feats_t[0]
    ftpk = jnp.concatenate((ft[:CH], ft[CH:]), axis=2)
    fr = feats_r[:, 0]
    frp = jnp.pad(fr, ((0, 0), (0, 0), (R, R + 4), (R, R)))
    q = quantized_r[:, 0, 0, ::D, ::D].astype(jnp.int32)
    labp = jnp.pad(q, ((0, 0), (R, R + 4), (R, R)),
                   constant_values=-1)

    nsearch = jnp.sum((current_ind - ref_index) > DIL)
    keep = jnp.arange(nref) >= nsearch
    maskadd = jnp.where(keep, 0.0, -jnp.inf).astype(jnp.float32)
    maskadd = maskadd.reshape(nref, 1, 1)

    out = pl.pallas_call(
        _colorizer_body,
        out_shape=jax.ShapeDtypeStruct((1, C, hh, ww), jnp.float32),
        scratch_shapes=[
            pltpu.VMEM((CH, ROWS, W2), jnp.float32),
            pltpu.VMEM((OFF, CH, WIN, W2), jnp.float32),
            pltpu.VMEM((P, CH, ROWS, W2), jnp.float32),
            pltpu.VMEM((P, C, ROWS, W2), jnp.float32),
            pltpu.VMEM((N, WIN, W2), jnp.float32),
            pltpu.VMEM((P, WIN, W2), jnp.float32),
            pltpu.VMEM((C, HH, W2), jnp.float32),
            pltpu.VMEM((1, HH, W2), jnp.float32),
            pltpu.VMEM((WIN, W2), jnp.float32),
        ],
    )(maskadd, ftpk, frp, labp)
    return out


# speedup vs baseline: 1.1200x; 1.0122x over previous
import jax
import jax.numpy as jnp
from jax.experimental import pallas as pl
from jax.experimental.pallas import tpu as pltpu

D = 4
R = 6
C = 16
P = 2 * R + 1
N = P * P
DIL = 15
HH = 56
WW = 56
CF = 64
NREF = 2
W2 = 2 * WW
CH = CF // 2
WIN = 64
ROWS = HH + 2 * R + 4
OFF = 8

DXG = ((0, 4), (4, 8), (8, 12), (12, 13))


def _colorizer_body(maskadd_ref, ftpk_ref, frp_ref, labp_ref, out_ref,
                    ftpad_ref, ftsall_ref, frxp_ref, labm_ref, corr_ref,
                    eu_ref, acc_ref, z_ref, tmp_ref):
    ftpad_ref[...] = jnp.zeros((CH, ROWS, W2), dtype=jnp.float32)
    ftpad_ref[:, OFF:OFF + HH, :] = ftpk_ref[...]
    for s in range(1, OFF + 1):
        ftsall_ref[s - 1] = ftpad_ref[:, s:s + WIN, :]

    for dx in range(P):
        sh = labp_ref[:, :, dx:dx + WW]
        for c in range(C):
            labm_ref[dx, c, :, :WW] = (sh[0] == c).astype(jnp.float32)
            labm_ref[dx, c, :, WW:] = (sh[1] == c).astype(jnp.float32)

    for i in range(NREF):
        for dx in range(P):
            frxp_ref[dx, :, :, :WW] = frp_ref[i, :CH, :, dx:dx + WW]
            frxp_ref[dx, :, :, WW:] = frp_ref[i, CH:CF, :, dx:dx + WW]

        madd = maskadd_ref[i, 0, 0]

        def corr_body(dy, _, i=i, madd=madd, base=0):
            sidx = OFF - 1 + base - dy
            for g0, g1 in DXG:
                accs = [None] * (g1 - g0)
                for cc in range(CH):
                    f = ftsall_ref[sidx, cc]
                    for j, dx in enumerate(range(g0, g1)):
                        t = f * frxp_ref[dx, cc, base:base + WIN, :]
                        accs[j] = t if cc == 0 else accs[j] + t
                for j, dx in enumerate(range(g0, g1)):
                    res = accs[j]
                    fold = res[:, :WW] + res[:, WW:] + madd
                    corr_ref[dy * P + dx, :, i * WW:(i + 1) * WW] = fold
            return 0

        from functools import partial
        jax.lax.fori_loop(0, OFF, partial(corr_body, i=i, madd=madd, base=0),
                          0, unroll=2)
        jax.lax.fori_loop(OFF, P, partial(corr_body, i=i, madd=madd, base=OFF),
                          0, unroll=False)

    acc_ref[...] = jnp.zeros((C, HH, W2), dtype=jnp.float32)
    z_ref[0] = jnp.zeros((HH, W2), dtype=jnp.float32)

    def acc_body(dy, _, base=0):
        o = dy - base
        zsum = None
        for dx in range(P):
            e = jnp.exp(corr_ref[dy * P + dx])
            eu_ref[dx] = e
            zsum = e if dx == 0 else zsum + e
        tmp_ref[...] = zsum
        z_ref[0] += tmp_ref[pl.ds(o, HH), :]
        for c in range(C):
            s = None
            for dx in range(P):
                t = eu_ref[dx] * labm_ref[dx, c, base:base + WIN, :]
                s = t if dx == 0 else s + t
            tmp_ref[...] = s
            acc_ref[c] += tmp_ref[pl.ds(o, HH), :]
        return 0

    from functools import partial as _pt
    jax.lax.fori_loop(0, OFF, _pt(acc_body, base=0), 0, unroll=2)
    jax.lax.fori_loop(OFF, P, _pt(acc_body, base=OFF), 0, unroll=False)

    z = z_ref[0]
    inv = 1.0 / (z[:, :WW] + z[:, WW:])
    for c in range(C):
        a = acc_ref[c]
        out_ref[0, c] = (a[:, :WW] + a[:, WW:]) * inv


def kernel(feats_r, feats_t, quantized_r, ref_index, current_ind):
    nref, b, cf, hh, ww = feats_r.shape
    ft = feats_t[0]
    ftpk = jnp.concatenate((ft[:CH], ft[CH:]), axis=2)
    fr = feats_r[:, 0]
    frp = jnp.pad(fr, ((0, 0), (0, 0), (R, R + 4), (R, R)))
    q = quantized_r[:, 0, 0, ::D, ::D].astype(jnp.int32)
    labp = jnp.pad(q, ((0, 0), (R, R + 4), (R, R)),
                   constant_values=-1)

    nsearch = jnp.sum((current_ind - ref_index) > DIL)
    keep = jnp.arange(nref) >= nsearch
    maskadd = jnp.where(keep, 0.0, -jnp.inf).astype(jnp.float32)
    maskadd = maskadd.reshape(nref, 1, 1)

    out = pl.pallas_call(
        _colorizer_body,
        out_shape=jax.ShapeDtypeStruct((1, C, hh, ww), jnp.float32),
        scratch_shapes=[
            pltpu.VMEM((CH, ROWS, W2), jnp.float32),
            pltpu.VMEM((OFF, CH, WIN, W2), jnp.float32),
            pltpu.VMEM((P, CH, ROWS, W2), jnp.float32),
            pltpu.VMEM((P, C, ROWS, W2), jnp.float32),
            pltpu.VMEM((N, WIN, W2), jnp.float32),
            pltpu.VMEM((P, WIN, W2), jnp.float32),
            pltpu.VMEM((C, HH, W2), jnp.float32),
            pltpu.VMEM((1, HH, W2), jnp.float32),
            pltpu.VMEM((WIN, W2), jnp.float32),
        ],
    )(maskadd, ftpk, frp, labp)
    return out


# speedup vs baseline: 1.1552x; 1.0315x over previous
import jax
import jax.numpy as jnp
from jax.experimental import pallas as pl
from jax.experimental.pallas import tpu as pltpu

D = 4
R = 6
C = 16
P = 2 * R + 1
N = P * P
DIL = 15
HH = 56
WW = 56
CF = 64
NREF = 2
W2 = 2 * WW
CH = CF // 2
WIN = 64
ROWS = HH + 2 * R + 4
OFF = 8

DXG = ((0, 2), (2, 4), (4, 6), (6, 8), (8, 10), (10, 12), (12, 13))


def _colorizer_body(maskadd_ref, ftpk_ref, frp_ref, labp_ref, out_ref,
                    ftpad_ref, ftsall_ref, frxp_ref, labm_ref, corr_ref,
                    eu_ref, acc_ref, z_ref, tmp_ref):
    ftpad_ref[...] = jnp.zeros((CH, ROWS, W2), dtype=jnp.float32)
    ftpad_ref[:, OFF:OFF + HH, :] = ftpk_ref[...]
    for s in range(1, OFF + 1):
        ftsall_ref[s - 1] = ftpad_ref[:, s:s + WIN, :]

    for dx in range(P):
        sh = labp_ref[:, :, dx:dx + WW]
        for c in range(C):
            labm_ref[dx, c, :, :WW] = (sh[0] == c).astype(jnp.float32)
            labm_ref[dx, c, :, WW:] = (sh[1] == c).astype(jnp.float32)

    for i in range(NREF):
        for dx in range(P):
            frxp_ref[dx, :, :, :WW] = frp_ref[i, :CH, :, dx:dx + WW]
            frxp_ref[dx, :, :, WW:] = frp_ref[i, CH:CF, :, dx:dx + WW]

        madd = maskadd_ref[i, 0, 0]

        def corr_body(dy, _, i=i, madd=madd, base=0):
            sidx = OFF - 1 + base - dy
            for g0, g1 in DXG:
                accs = [None] * (g1 - g0)
                for cc in range(CH):
                    f = ftsall_ref[sidx, cc]
                    for j, dx in enumerate(range(g0, g1)):
                        t = f * frxp_ref[dx, cc, base:base + WIN, :]
                        accs[j] = t if cc == 0 else accs[j] + t
                for j, dx in enumerate(range(g0, g1)):
                    res = accs[j]
                    fold = res[:, :WW] + res[:, WW:] + madd
                    corr_ref[dy * P + dx, :, i * WW:(i + 1) * WW] = fold
            return 0

        from functools import partial
        jax.lax.fori_loop(0, OFF, partial(corr_body, i=i, madd=madd, base=0),
                          0, unroll=2)
        jax.lax.fori_loop(OFF, P, partial(corr_body, i=i, madd=madd, base=OFF),
                          0, unroll=False)

    acc_ref[...] = jnp.zeros((C, HH, W2), dtype=jnp.float32)
    z_ref[0] = jnp.zeros((HH, W2), dtype=jnp.float32)

    def acc_body(dy, _, base=0):
        o = dy - base
        zsum = None
        for dx in range(P):
            e = jnp.exp(corr_ref[dy * P + dx])
            eu_ref[dx] = e
            zsum = e if dx == 0 else zsum + e
        tmp_ref[...] = zsum
        z_ref[0] += tmp_ref[pl.ds(o, HH), :]
        for c in range(C):
            s = None
            for dx in range(P):
                t = eu_ref[dx] * labm_ref[dx, c, base:base + WIN, :]
                s = t if dx == 0 else s + t
            tmp_ref[...] = s
            acc_ref[c] += tmp_ref[pl.ds(o, HH), :]
        return 0

    from functools import partial as _pt
    jax.lax.fori_loop(0, OFF, _pt(acc_body, base=0), 0, unroll=2)
    jax.lax.fori_loop(OFF, P, _pt(acc_body, base=OFF), 0, unroll=False)

    z = z_ref[0]
    inv = 1.0 / (z[:, :WW] + z[:, WW:])
    for c in range(C):
        a = acc_ref[c]
        out_ref[0, c] = (a[:, :WW] + a[:, WW:]) * inv


def kernel(feats_r, feats_t, quantized_r, ref_index, current_ind):
    nref, b, cf, hh, ww = feats_r.shape
    ft = feats_t[0]
    ftpk = jnp.concatenate((ft[:CH], ft[CH:]), axis=2)
    fr = feats_r[:, 0]
    frp = jnp.pad(fr, ((0, 0), (0, 0), (R, R + 4), (R, R)))
    q = quantized_r[:, 0, 0, ::D, ::D].astype(jnp.int32)
    labp = jnp.pad(q, ((0, 0), (R, R + 4), (R, R)),
                   constant_values=-1)

    nsearch = jnp.sum((current_ind - ref_index) > DIL)
    keep = jnp.arange(nref) >= nsearch
    maskadd = jnp.where(keep, 0.0, -jnp.inf).astype(jnp.float32)
    maskadd = maskadd.reshape(nref, 1, 1)

    out = pl.pallas_call(
        _colorizer_body,
        out_shape=jax.ShapeDtypeStruct((1, C, hh, ww), jnp.float32),
        scratch_shapes=[
            pltpu.VMEM((CH, ROWS, W2), jnp.float32),
            pltpu.VMEM((OFF, CH, WIN, W2), jnp.float32),
            pltpu.VMEM((P, CH, ROWS, W2), jnp.float32),
            pltpu.VMEM((P, C, ROWS, W2), jnp.float32),
            pltpu.VMEM((N, WIN, W2), jnp.float32),
            pltpu.VMEM((P, WIN, W2), jnp.float32),
            pltpu.VMEM((C, HH, W2), jnp.float32),
            pltpu.VMEM((1, HH, W2), jnp.float32),
            pltpu.VMEM((WIN, W2), jnp.float32),
        ],
    )(maskadd, ftpk, frp, labp)
    return out
